# baseline (speedup 1.0000x reference)
import jax
import jax.numpy as jnp
from jax import lax
from jax.experimental import pallas as pl
from jax.experimental.pallas import tpu as pltpu
from jax.experimental.pallas import tpu_sc as plsc

_S0, _S1 = 2048, 1024
_O0, _O1 = 4096, 1024
_NA = 8
_BATCH = 128
_NW = 32
_EPW = _BATCH // _NW

_GDN = lax.GatherDimensionNumbers(offset_dims=(), collapsed_slice_dims=(0,),
                                  start_index_map=(0,))


def _bcast(v, k):
    return lax.gather(v, jnp.full((16, 1), k, jnp.int32), _GDN,
                      slice_sizes=(1,),
                      mode=lax.GatherScatterMode.PROMISE_IN_BOUNDS)


def _excl_scan16(t, init=None):
    lanes = lax.iota(jnp.int32, 16)
    off = jnp.zeros((16,), jnp.float32) if init is None else init
    for k in range(15):
        off = jnp.where(lanes > k, off + _bcast(t, k), off)
    return off


def _cnt(mask, cnt, one, zero):
    return cnt + jnp.where(mask, one, zero)


def _lanesum(v, n):
    tot = _bcast(v, 0)
    for k in range(1, n):
        tot = tot + _bcast(v, k)
    return tot


def _sc_body(B0f, B1f, A0f, A1f, idxB0_h, idxB1_h, abase0_h, abase1_h,
             omu_h, o0_h, o1_h, s0_h, s1_h,
             idxB0_s0, idxB0_s1, idxB0_s2, idxB0_s3,
             idxB1_s0, idxB1_s1, idxB1_s2, idxB1_s3,
             dstB0_s0, dstB0_s1, dstB0_s2, dstB0_s3,
             dstB1_s0, dstB1_s1, dstB1_s2, dstB1_s3,
             abase0_s, abase1_s,
             idxA0_s0, idxA0_s1, idxA0_s2, idxA0_s3,
             idxA1_s0, idxA1_s1, idxA1_s2, idxA1_s3,
             dstA0_s0, dstA0_s1, dstA0_s2, dstA0_s3,
             dstA1_s0, dstA1_s1, dstA1_s2, dstA1_s3,
             omu_s, res_s0, res_s1, res_s2, res_s3,
             semB, semA):
    idxB0_s = [idxB0_s0, idxB0_s1, idxB0_s2, idxB0_s3]
    idxB1_s = [idxB1_s0, idxB1_s1, idxB1_s2, idxB1_s3]
    dstB0_s = [dstB0_s0, dstB0_s1, dstB0_s2, dstB0_s3]
    dstB1_s = [dstB1_s0, dstB1_s1, dstB1_s2, dstB1_s3]
    idxA0_s = [idxA0_s0, idxA0_s1, idxA0_s2, idxA0_s3]
    idxA1_s = [idxA1_s0, idxA1_s1, idxA1_s2, idxA1_s3]
    dstA0_s = [dstA0_s0, dstA0_s1, dstA0_s2, dstA0_s3]
    dstA1_s = [dstA1_s0, dstA1_s1, dstA1_s2, dstA1_s3]
    res_s = [res_s0, res_s1, res_s2, res_s3]

    lanes = lax.iota(jnp.int32, 16)
    one = jnp.full((16,), 1, jnp.int32)
    zero = jnp.zeros((16,), jnp.int32)

    wid = lax.axis_index("s") * 2 + lax.axis_index("c")

    pltpu.sync_copy(omu_h.at[wid], omu_s)
    pltpu.sync_copy(abase0_h, abase0_s)
    pltpu.sync_copy(abase1_h, abase1_s)
    omu_v = omu_s[:]

    for e in range(_EPW):
        r = wid * _EPW + e
        pltpu.sync_copy(idxB0_h.at[r], idxB0_s[e])
        pltpu.sync_copy(idxB1_h.at[r], idxB1_s[e])
    cps = []
    for e in range(_EPW):
        cps.append(pltpu.async_copy(B0f.at[idxB0_s[e]], dstB0_s[e], semB))
        cps.append(pltpu.async_copy(B1f.at[idxB1_s[e]], dstB1_s[e], semB))
    for cp in cps:
        cp.wait()

    def b1_body(j, c):
        a0, a1 = c
        p = j * 16
        a0 = tuple(a0[e] + dstB0_s[e][pl.ds(p, 16)] for e in range(_EPW))
        a1 = tuple(a1[e] + dstB1_s[e][pl.ds(p, 16)] for e in range(_EPW))
        return (a0, a1)

    t0, t1 = lax.fori_loop(
        1, 128, b1_body,
        (tuple(dstB0_s[e][pl.ds(0, 16)] for e in range(_EPW)),
         tuple(dstB1_s[e][pl.ds(0, 16)] for e in range(_EPW))))

    offs0, offs1, rv0, rv1 = [], [], [], []
    for e in range(_EPW):
        off = _excl_scan16(t0[e])
        offs0.append(off)
        rv0.append(_bcast(off + t0[e], 15) * _bcast(omu_v, e))
        off = _excl_scan16(t1[e])
        offs1.append(off)
        rv1.append(_bcast(off + t1[e], 7) * _bcast(omu_v, 4 + e))

    def b2_body(j, c):
        a0, a1, c0, c1 = c
        p = j * 16
        na0, na1, nc0, nc1 = [], [], [], []
        for e in range(_EPW):
            acc = a0[e] + dstB0_s[e][pl.ds(p, 16)]
            na0.append(acc)
            nc0.append(_cnt((acc + offs0[e]) < rv0[e], c0[e], one, zero))
            acc = a1[e] + dstB1_s[e][pl.ds(p, 16)]
            na1.append(acc)
            nc1.append(_cnt((acc + offs1[e]) < rv1[e], c1[e], one, zero))
        return (tuple(na0), tuple(na1), tuple(nc0), tuple(nc1))

    ia0, ia1, ic0, ic1 = [], [], [], []
    for e in range(_EPW):
        acc = dstB0_s[e][pl.ds(0, 16)]
        ia0.append(acc)
        ic0.append(jnp.where((acc + offs0[e]) < rv0[e], one, zero))
        acc = dstB1_s[e][pl.ds(0, 16)]
        ia1.append(acc)
        ic1.append(jnp.where((acc + offs1[e]) < rv1[e], one, zero))
    _, _, c0, c1 = lax.fori_loop(
        1, 128, b2_body, (tuple(ia0), tuple(ia1), tuple(ic0), tuple(ic1)))

    s0spl = [_lanesum(c0[e], 16) for e in range(_EPW)]
    s1spl = [_lanesum(c1[e], 8) for e in range(_EPW)]

    def idx_body(j, c):
        p32 = j * 32
        p16 = j * 16
        lo = abase0_s[pl.ds(p32, 16)]
        hi = abase0_s[pl.ds(p32 + 16, 16)]
        a1row = abase1_s[pl.ds(p16, 16)]
        for e in range(_EPW):
            idxA0_s[e][pl.ds(p32, 16)] = lo + s0spl[e]
            idxA0_s[e][pl.ds(p32 + 16, 16)] = hi + s0spl[e]
            idxA1_s[e][pl.ds(p16, 16)] = a1row + s1spl[e]
        return c

    lax.fori_loop(0, 128, idx_body, 0)

    cps = []
    for e in range(_EPW):
        cps.append(pltpu.async_copy(A0f.at[idxA0_s[e]], dstA0_s[e], semA))
        cps.append(pltpu.async_copy(A1f.at[idxA1_s[e]], dstA1_s[e], semA))
    for cp in cps:
        cp.wait()

    def a1_body(j, c):
        alo, ahi, a1 = c
        p32 = j * 32
        p16 = j * 16
        alo = tuple(alo[e] + dstA0_s[e][pl.ds(p32, 16)] for e in range(_EPW))
        ahi = tuple(ahi[e] + dstA0_s[e][pl.ds(p32 + 16, 16)]
                    for e in range(_EPW))
        a1 = tuple(a1[e] + dstA1_s[e][pl.ds(p16, 16)] for e in range(_EPW))
        return (alo, ahi, a1)

    tlo, thi, tA1 = lax.fori_loop(
        1, 128, a1_body,
        (tuple(dstA0_s[e][pl.ds(0, 16)] for e in range(_EPW)),
         tuple(dstA0_s[e][pl.ds(16, 16)] for e in range(_EPW)),
         tuple(dstA1_s[e][pl.ds(0, 16)] for e in range(_EPW))))

    offlo, offhi, rvA0, offA1, rvA1 = [], [], [], [], []
    for e in range(_EPW):
        olo = _excl_scan16(tlo[e])
        bridge = _bcast(olo + tlo[e], 15)
        ohi = _excl_scan16(thi[e], init=bridge)
        offlo.append(olo)
        offhi.append(ohi)
        rvA0.append(_bcast(ohi + thi[e], 15) * _bcast(omu_v, 8 + e))
        oa1 = _excl_scan16(tA1[e])
        offA1.append(oa1)
        rvA1.append(_bcast(oa1 + tA1[e], 7) * _bcast(omu_v, 12 + e))

    def a2_body(j, c):
        alo, ahi, a1, clo, chi, c1 = c
        p32 = j * 32
        p16 = j * 16
        nalo, nahi, na1, nclo, nchi, nc1 = [], [], [], [], [], []
        for e in range(_EPW):
            acc = alo[e] + dstA0_s[e][pl.ds(p32, 16)]
            nalo.append(acc)
            nclo.append(_cnt((acc + offlo[e]) < rvA0[e], clo[e], one, zero))
            acc = ahi[e] + dstA0_s[e][pl.ds(p32 + 16, 16)]
            nahi.append(acc)
            nchi.append(_cnt((acc + offhi[e]) < rvA0[e], chi[e], one, zero))
            acc = a1[e] + dstA1_s[e][pl.ds(p16, 16)]
            na1.append(acc)
            nc1.append(_cnt((acc + offA1[e]) < rvA1[e], c1[e], one, zero))
        return (tuple(nalo), tuple(nahi), tuple(na1),
                tuple(nclo), tuple(nchi), tuple(nc1))

    ialo, iahi, ia1, iclo, ichi, ic1 = [], [], [], [], [], []
    for e in range(_EPW):
        acc = dstA0_s[e][pl.ds(0, 16)]
        ialo.append(acc)
        iclo.append(jnp.where((acc + offlo[e]) < rvA0[e], one, zero))
        acc = dstA0_s[e][pl.ds(16, 16)]
        iahi.append(acc)
        ichi.append(jnp.where((acc + offhi[e]) < rvA0[e], one, zero))
        acc = dstA1_s[e][pl.ds(0, 16)]
        ia1.append(acc)
        ic1.append(jnp.where((acc + offA1[e]) < rvA1[e], one, zero))
    _, _, _, clo, chi, c1 = lax.fori_loop(
        1, 128, a2_body,
        (tuple(ialo), tuple(iahi), tuple(ia1),
         tuple(iclo), tuple(ichi), tuple(ic1)))

    o0 = [_lanesum(clo[e], 16) + _lanesum(chi[e], 16) for e in range(_EPW)]
    o1 = [_lanesum(c1[e], 8) for e in range(_EPW)]

    for res, vals in ((res_s0, o0), (res_s1, o1), (res_s2, s0spl),
                      (res_s3, s1spl)):
        v = zero
        for e in range(_EPW):
            v = jnp.where(lanes == e, vals[e], v)
        res[:] = v
    pltpu.sync_copy(res_s0, o0_h.at[wid])
    pltpu.sync_copy(res_s1, o1_h.at[wid])
    pltpu.sync_copy(res_s2, s0_h.at[wid])
    pltpu.sync_copy(res_s3, s1_h.at[wid])


def kernel(B0, B1, A0, A1, states0, states1, actions):
    key = jax.random.key(1234)
    key_state, key_obs = jax.random.split(key)
    ks0, ks1 = jax.random.split(key_state)
    ko0, ko1 = jax.random.split(key_obs)

    def _u(k):
        return jax.vmap(lambda kk: jax.random.uniform(kk, (), jnp.float32))(
            jax.random.split(k, _BATCH))

    omu = [1.0 - _u(k) for k in (ks0, ks1, ko0, ko1)]
    omu_pack = jnp.stack([m.reshape(_NW, _EPW) for m in omu],
                         axis=1).reshape(_NW, 16)

    j128 = jnp.arange(128, dtype=jnp.int32)
    lane16 = jnp.arange(16, dtype=jnp.int32)
    iB0 = 128 * lane16[None, :] + j128[:, None]
    baseB0 = states0.astype(jnp.int32) * _NA + actions[:, 0].astype(jnp.int32)
    idxB0 = (iB0[None] * (_S0 * _NA)
             + baseB0[:, None, None]).reshape(_BATCH, 2048)
    iB1 = 128 * lane16[None, :] + j128[:, None]
    baseB1 = states1.astype(jnp.int32) * _NA + actions[:, 1].astype(jnp.int32)
    idxB1 = jnp.where(lane16[None, None, :] < 8,
                      iB1[None] * (_S1 * _NA) + baseB1[:, None, None],
                      0).reshape(_BATCH, 2048)
    c32 = jnp.arange(32, dtype=jnp.int32)
    abase0 = ((128 * c32[None, :] + j128[:, None]) * _S0).reshape(4096)
    abase1 = jnp.where(lane16[None, :] < 8,
                       (128 * lane16[None, :] + j128[:, None]) * _S1,
                       0).reshape(2048)

    mesh = plsc.VectorSubcoreMesh(core_axis_name="c", subcore_axis_name="s")
    out_t = [jax.ShapeDtypeStruct((_NW, 16), jnp.int32)] * 4
    scratch = (
        [pltpu.VMEM((2048,), jnp.int32)] * 4 +
        [pltpu.VMEM((2048,), jnp.int32)] * 4 +
        [pltpu.VMEM((2048,), jnp.float32)] * 4 +
        [pltpu.VMEM((2048,), jnp.float32)] * 4 +
        [pltpu.VMEM((4096,), jnp.int32),
         pltpu.VMEM((2048,), jnp.int32)] +
        [pltpu.VMEM((4096,), jnp.int32)] * 4 +
        [pltpu.VMEM((2048,), jnp.int32)] * 4 +
        [pltpu.VMEM((4096,), jnp.float32)] * 4 +
        [pltpu.VMEM((2048,), jnp.float32)] * 4 +
        [pltpu.VMEM((16,), jnp.float32)] +
        [pltpu.VMEM((16,), jnp.int32)] * 4 +
        [pltpu.SemaphoreType.DMA] * 2)

    run = pl.kernel(_sc_body, out_type=out_t, mesh=mesh,
                    scratch_types=scratch)
    o0p, o1p, s0p, s1p = run(
        B0.reshape(-1), B1.reshape(-1), A0.reshape(-1), A1.reshape(-1),
        idxB0, idxB1, abase0, abase1, omu_pack)

    unpack = lambda x: x[:, :_EPW].reshape(_BATCH)
    return (unpack(o0p), unpack(o1p), unpack(s0p), unpack(s1p))
